# baseline (speedup 1.0000x reference)
import functools

import jax
import jax.numpy as jnp
from jax import lax
from jax.experimental import pallas as pl
from jax.experimental.pallas import tpu as pltpu
from jax.experimental.pallas import tpu_sc as plsc

NUM_ACTIONS = 100000
D_MODEL = 128
HALF = D_MODEL // 2
B = 4096
L = 200
TOKENS = B * L

NW = 32
CHUNK = 128
ROWS_PER_W = TOKENS // NW
CHUNKS = ROWS_PER_W // CHUNK

VPAD = 102400


def _fuse_table(table_pad, Wo_top):
    RBLK = 2048
    grid = (VPAD // RBLK,)

    def body(t_ref, w_ref, o_ref):
        o_ref[...] = jnp.dot(
            t_ref[...], w_ref[...], preferred_element_type=jnp.float32
        )

    return pl.pallas_call(
        body,
        grid=grid,
        in_specs=[
            pl.BlockSpec((RBLK, HALF), lambda i: (i, 0)),
            pl.BlockSpec((HALF, D_MODEL), lambda i: (0, 0)),
        ],
        out_specs=pl.BlockSpec((RBLK, D_MODEL), lambda i: (i, 0)),
        out_shape=jax.ShapeDtypeStruct((VPAD, D_MODEL), jnp.float32),
    )(table_pad, Wo_top)


def _sc_gather(types32, table2):
    mesh = plsc.VectorSubcoreMesh(core_axis_name="c", subcore_axis_name="s")

    @functools.partial(
        pl.kernel,
        out_type=jax.ShapeDtypeStruct((TOKENS, D_MODEL), jnp.float32),
        mesh=mesh,
        scratch_types=[
            pltpu.VMEM((CHUNKS, CHUNK), jnp.int32),
            pltpu.VMEM((CHUNK, D_MODEL), jnp.float32),
            pltpu.SemaphoreType.DMA,
        ],
    )
    def gather_kernel(idx_hbm, table_hbm, out_hbm, idx_v, rows_v, sem):
        wid = lax.axis_index("s") * 2 + lax.axis_index("c")
        base = wid * ROWS_PER_W
        pltpu.sync_copy(idx_hbm.at[wid], idx_v)

        def body(j, _):
            pltpu.async_copy(table_hbm.at[idx_v.at[j]], rows_v, sem).wait()
            pltpu.sync_copy(rows_v, out_hbm.at[pl.ds(base + j * CHUNK, CHUNK)])
            return 0

        lax.fori_loop(0, CHUNKS, body, 0)

    return gather_kernel(types32, table2)


def _tc_tail(e2, cont2d, Wc, bc2, Wo, bo2, gamma2, beta2):
    TBLK = 2048
    grid = (TOKENS // TBLK,)

    def body(e_ref, c_ref, wc_ref, bc_ref, wo_ref, bo_ref, g_ref, b_ref, o_ref):
        wo_bot = wo_ref[...][HALF:, :]
        w2 = jnp.dot(wc_ref[...], wo_bot, preferred_element_type=jnp.float32)
        b2 = (
            jnp.dot(bc_ref[...], wo_bot, preferred_element_type=jnp.float32)
            + bo_ref[...]
        )
        o = (
            e_ref[...]
            + jnp.dot(c_ref[...], w2, preferred_element_type=jnp.float32)
            + b2
        )
        mu = jnp.mean(o, axis=-1, keepdims=True)
        d = o - mu
        var = jnp.mean(d * d, axis=-1, keepdims=True)
        y = d * lax.rsqrt(var + 1e-5)
        o_ref[...] = y * g_ref[...] + b_ref[...]

    def wspec(shape):
        return pl.BlockSpec(shape, lambda i: (0, 0))

    return pl.pallas_call(
        body,
        grid=grid,
        in_specs=[
            pl.BlockSpec((TBLK, D_MODEL), lambda i: (i, 0)),
            pl.BlockSpec((TBLK, 3), lambda i: (i, 0)),
            wspec((3, HALF)),
            wspec((1, HALF)),
            wspec((D_MODEL, D_MODEL)),
            wspec((1, D_MODEL)),
            wspec((1, D_MODEL)),
            wspec((1, D_MODEL)),
        ],
        out_specs=pl.BlockSpec((TBLK, D_MODEL), lambda i: (i, 0)),
        out_shape=jax.ShapeDtypeStruct((TOKENS, D_MODEL), jnp.float32),
    )(e2, cont2d, Wc, bc2, Wo, bo2, gamma2, beta2)


def kernel(types, cont, table, Wc, bc, Wo, bo, gamma, beta):
    types32 = types.astype(jnp.int32).reshape(NW, CHUNKS, CHUNK)
    table_pad = jnp.zeros((VPAD, HALF), jnp.float32).at[: NUM_ACTIONS + 1].set(table)
    table2 = _fuse_table(table_pad, Wo[:HALF, :])
    e2 = _sc_gather(types32, table2)
    cont2d = cont.reshape(TOKENS, 3)
    out = _tc_tail(
        e2,
        cont2d,
        Wc,
        bc.reshape(1, HALF),
        Wo,
        bo.reshape(1, D_MODEL),
        gamma.reshape(1, D_MODEL),
        beta.reshape(1, D_MODEL),
    )
    return out.reshape(B, L, D_MODEL)


# speedup vs baseline: 3.2714x; 3.2714x over previous
import functools

import jax
import jax.numpy as jnp
from jax import lax
from jax.experimental import pallas as pl
from jax.experimental.pallas import tpu as pltpu
from jax.experimental.pallas import tpu_sc as plsc

NUM_ACTIONS = 100000
D_MODEL = 128
HALF = D_MODEL // 2
B = 4096
L = 200
TOKENS = B * L

NW = 32
CHUNK = 128
ROWS_PER_W = TOKENS // NW
CHUNKS = ROWS_PER_W // CHUNK

VPAD = NUM_ACTIONS + 1


def _fuse_table(tableT, Wo_top):
    RBLK = 2048
    grid = (pl.cdiv(VPAD, RBLK),)

    def body(t_ref, w_ref, o_ref):
        o_ref[...] = lax.dot_general(
            t_ref[...],
            w_ref[...],
            ((( 0,), (0,)), ((), ())),
            preferred_element_type=jnp.float32,
        )

    return pl.pallas_call(
        body,
        grid=grid,
        in_specs=[
            pl.BlockSpec((HALF, RBLK), lambda i: (0, i)),
            pl.BlockSpec((HALF, D_MODEL), lambda i: (0, 0)),
        ],
        out_specs=pl.BlockSpec((RBLK, D_MODEL), lambda i: (i, 0)),
        out_shape=jax.ShapeDtypeStruct((VPAD, D_MODEL), jnp.float32),
    )(tableT, Wo_top)


def _sc_gather(types32, table2):
    mesh = plsc.VectorSubcoreMesh(core_axis_name="c", subcore_axis_name="s")

    @functools.partial(
        pl.kernel,
        out_type=jax.ShapeDtypeStruct((TOKENS, D_MODEL), jnp.float32),
        mesh=mesh,
        scratch_types=[
            pltpu.VMEM((CHUNKS, CHUNK), jnp.int32),
            pltpu.VMEM((CHUNK, D_MODEL), jnp.float32),
            pltpu.SemaphoreType.DMA,
        ],
    )
    def gather_kernel(idx_hbm, table_hbm, out_hbm, idx_v, rows_v, sem):
        wid = lax.axis_index("s") * 2 + lax.axis_index("c")
        base = wid * ROWS_PER_W
        pltpu.sync_copy(idx_hbm.at[wid], idx_v)

        def body(j, _):
            pltpu.async_copy(table_hbm.at[idx_v.at[j]], rows_v, sem).wait()
            pltpu.sync_copy(rows_v, out_hbm.at[pl.ds(base + j * CHUNK, CHUNK)])
            return 0

        lax.fori_loop(0, CHUNKS, body, 0)

    return gather_kernel(types32, table2)


def _tc_tail(e2, cont3, Wc, bc2, Wo, bo2, gamma2, beta2):
    TBLK = 2048
    grid = (TOKENS // TBLK,)

    def body(e_ref, c_ref, wc_ref, bc_ref, wo_ref, bo_ref, g_ref, b_ref, o_ref):
        wo_bot = wo_ref[...][HALF:, :]
        w2 = jnp.dot(wc_ref[...], wo_bot, preferred_element_type=jnp.float32)
        b2 = (
            jnp.dot(bc_ref[...], wo_bot, preferred_element_type=jnp.float32)
            + bo_ref[...]
        )
        o = (
            e_ref[...]
            + lax.dot_general(
                c_ref[...],
                w2,
                (((0,), (0,)), ((), ())),
                preferred_element_type=jnp.float32,
            )
            + b2
        )
        mu = jnp.mean(o, axis=-1, keepdims=True)
        d = o - mu
        var = jnp.mean(d * d, axis=-1, keepdims=True)
        y = d * lax.rsqrt(var + 1e-5)
        o_ref[...] = y * g_ref[...] + b_ref[...]

    def wspec(shape):
        return pl.BlockSpec(shape, lambda i: (0, 0))

    return pl.pallas_call(
        body,
        grid=grid,
        in_specs=[
            pl.BlockSpec((TBLK, D_MODEL), lambda i: (i, 0)),
            pl.BlockSpec((3, TBLK), lambda i: (0, i)),
            wspec((3, HALF)),
            wspec((1, HALF)),
            wspec((D_MODEL, D_MODEL)),
            wspec((1, D_MODEL)),
            wspec((1, D_MODEL)),
            wspec((1, D_MODEL)),
        ],
        out_specs=pl.BlockSpec((TBLK, D_MODEL), lambda i: (i, 0)),
        out_shape=jax.ShapeDtypeStruct((TOKENS, D_MODEL), jnp.float32),
    )(e2, cont3, Wc, bc2, Wo, bo2, gamma2, beta2)


def kernel(types, cont, table, Wc, bc, Wo, bo, gamma, beta):
    types32 = types.astype(jnp.int32).reshape(NW, CHUNKS, CHUNK)
    table2 = _fuse_table(table.T, Wo[:HALF, :])
    e2 = _sc_gather(types32, table2)
    cont3 = jnp.transpose(cont, (2, 0, 1)).reshape(3, TOKENS)
    out = _tc_tail(
        e2,
        cont3,
        Wc,
        bc.reshape(1, HALF),
        Wo,
        bo.reshape(1, D_MODEL),
        gamma.reshape(1, D_MODEL),
        beta.reshape(1, D_MODEL),
    )
    return out.reshape(B, L, D_MODEL)


# speedup vs baseline: 4.3019x; 1.3150x over previous
import functools

import jax
import jax.numpy as jnp
from jax import lax
from jax.experimental import pallas as pl
from jax.experimental.pallas import tpu as pltpu
from jax.experimental.pallas import tpu_sc as plsc

NUM_ACTIONS = 100000
D_MODEL = 128
HALF = D_MODEL // 2
B = 4096
L = 200
TOKENS = B * L

NW = 32
CHUNK = 128
ROWS_PER_W = TOKENS // NW
CHUNKS = ROWS_PER_W // CHUNK

VPAD = NUM_ACTIONS + 1


def _fuse_table(tableT, Wo_top):
    RBLK = 2048
    grid = (pl.cdiv(VPAD, RBLK),)

    def body(t_ref, w_ref, o_ref):
        o_ref[...] = lax.dot_general(
            t_ref[...],
            w_ref[...],
            ((( 0,), (0,)), ((), ())),
            preferred_element_type=jnp.float32,
        )

    return pl.pallas_call(
        body,
        grid=grid,
        in_specs=[
            pl.BlockSpec((HALF, RBLK), lambda i: (0, i)),
            pl.BlockSpec((HALF, D_MODEL), lambda i: (0, 0)),
        ],
        out_specs=pl.BlockSpec((RBLK, D_MODEL), lambda i: (i, 0)),
        out_shape=jax.ShapeDtypeStruct((VPAD, D_MODEL), jnp.float32),
    )(tableT, Wo_top)


def _sc_gather(types32, table2):
    mesh = plsc.VectorSubcoreMesh(core_axis_name="c", subcore_axis_name="s")

    NBUF = 4
    LOOK = 2

    @functools.partial(
        pl.kernel,
        out_type=jax.ShapeDtypeStruct((TOKENS, D_MODEL), jnp.float32),
        mesh=mesh,
        scratch_types=[
            pltpu.VMEM((CHUNKS, CHUNK), jnp.int32),
            [pltpu.VMEM((CHUNK, D_MODEL), jnp.float32) for _ in range(NBUF)],
            [pltpu.SemaphoreType.DMA for _ in range(NBUF)],
            [pltpu.SemaphoreType.DMA for _ in range(NBUF)],
        ],
    )
    def gather_kernel(idx_hbm, table_hbm, out_hbm, idx_v, bufs, gsem, osem):
        wid = lax.axis_index("s") * 2 + lax.axis_index("c")
        base = wid * ROWS_PER_W
        pltpu.sync_copy(idx_hbm.at[wid], idx_v)

        def gath(j, s):
            pltpu.async_copy(table_hbm.at[idx_v.at[j]], bufs[s], gsem[s])

        def gath_wait(j, s):
            pltpu.make_async_copy(table_hbm.at[idx_v.at[j]], bufs[s], gsem[s]).wait()

        def outc(j, s):
            pltpu.async_copy(
                bufs[s], out_hbm.at[pl.ds(base + j * CHUNK, CHUNK)], osem[s]
            )

        def outc_wait(j, s):
            pltpu.make_async_copy(
                bufs[s], out_hbm.at[pl.ds(base + j * CHUNK, CHUNK)], osem[s]
            ).wait()

        for j in range(LOOK):
            gath(j, j)
        for j in range(LOOK):
            gath(j + LOOK, j + LOOK)
            gath_wait(j, j)
            outc(j, j)

        def body(o, _):
            for k in range(NBUF):
                j = LOOK + o * NBUF + k
                s_next = k
                outc_wait(j - LOOK, s_next)
                gath(j + LOOK, s_next)
                s = (LOOK + k) % NBUF
                gath_wait(j, s)
                outc(j, s)
            return 0

        lax.fori_loop(0, (CHUNKS - 2 * LOOK) // NBUF, body, 0)

        for j in range(CHUNKS - LOOK, CHUNKS):
            s = j % NBUF
            gath_wait(j, s)
            outc(j, s)
        for s in range(NBUF):
            outc_wait(CHUNKS - NBUF + s, s)

    return gather_kernel(types32, table2)


def _tc_tail(e2, cont3, Wc, bc2, Wo, bo2, gamma2, beta2):
    TBLK = 4096
    grid = (TOKENS // TBLK,)

    def body(e_ref, c_ref, wc_ref, bc_ref, wo_ref, bo_ref, g_ref, b_ref, o_ref):
        wo_bot = wo_ref[...][HALF:, :]
        w2 = jnp.dot(wc_ref[...], wo_bot, preferred_element_type=jnp.float32)
        b2 = (
            jnp.dot(bc_ref[...], wo_bot, preferred_element_type=jnp.float32)
            + bo_ref[...]
        )
        o = (
            e_ref[...]
            + lax.dot_general(
                c_ref[...],
                w2,
                (((0,), (0,)), ((), ())),
                preferred_element_type=jnp.float32,
            )
            + b2
        )
        mu = jnp.mean(o, axis=-1, keepdims=True)
        d = o - mu
        var = jnp.mean(d * d, axis=-1, keepdims=True)
        y = d * lax.rsqrt(var + 1e-5)
        o_ref[...] = y * g_ref[...] + b_ref[...]

    def wspec(shape):
        return pl.BlockSpec(shape, lambda i: (0, 0))

    return pl.pallas_call(
        body,
        grid=grid,
        in_specs=[
            pl.BlockSpec((TBLK, D_MODEL), lambda i: (i, 0)),
            pl.BlockSpec((3, TBLK), lambda i: (0, i)),
            wspec((3, HALF)),
            wspec((1, HALF)),
            wspec((D_MODEL, D_MODEL)),
            wspec((1, D_MODEL)),
            wspec((1, D_MODEL)),
            wspec((1, D_MODEL)),
        ],
        out_specs=pl.BlockSpec((TBLK, D_MODEL), lambda i: (i, 0)),
        out_shape=jax.ShapeDtypeStruct((TOKENS, D_MODEL), jnp.float32),
    )(e2, cont3, Wc, bc2, Wo, bo2, gamma2, beta2)


def kernel(types, cont, table, Wc, bc, Wo, bo, gamma, beta):
    types32 = types.astype(jnp.int32).reshape(NW, CHUNKS, CHUNK)
    table2 = _fuse_table(table.T, Wo[:HALF, :])
    e2 = _sc_gather(types32, table2)
    cont3 = jnp.transpose(cont, (2, 0, 1)).reshape(3, TOKENS)
    out = _tc_tail(
        e2,
        cont3,
        Wc,
        bc.reshape(1, HALF),
        Wo,
        bo.reshape(1, D_MODEL),
        gamma.reshape(1, D_MODEL),
        beta.reshape(1, D_MODEL),
    )
    return out.reshape(B, L, D_MODEL)


# speedup vs baseline: 4.7950x; 1.1146x over previous
import functools

import jax
import jax.numpy as jnp
from jax import lax
from jax.experimental import pallas as pl
from jax.experimental.pallas import tpu as pltpu
from jax.experimental.pallas import tpu_sc as plsc

NUM_ACTIONS = 100000
D_MODEL = 128
HALF = D_MODEL // 2
B = 4096
L = 200
TOKENS = B * L

NCH = 4
CHTOK = TOKENS // NCH

NW = 32
CHUNK = 128
ROWS_PER_W = CHTOK // NW
CHUNKS = ROWS_PER_W // CHUNK

VPAD = NUM_ACTIONS + 1


def _fuse_table(tableT, Wo_top):
    RBLK = 2048
    grid = (pl.cdiv(VPAD, RBLK),)

    def body(t_ref, w_ref, o_ref):
        o_ref[...] = lax.dot_general(
            t_ref[...],
            w_ref[...],
            (((0,), (0,)), ((), ())),
            preferred_element_type=jnp.float32,
        )

    return pl.pallas_call(
        body,
        grid=grid,
        in_specs=[
            pl.BlockSpec((HALF, RBLK), lambda i: (0, i)),
            pl.BlockSpec((HALF, D_MODEL), lambda i: (0, 0)),
        ],
        out_specs=pl.BlockSpec((RBLK, D_MODEL), lambda i: (i, 0)),
        out_shape=jax.ShapeDtypeStruct((VPAD, D_MODEL), jnp.float32),
    )(tableT, Wo_top)


NBUF = 4
LOOK = 2


def _sc_gather(types32, table2):
    mesh = plsc.VectorSubcoreMesh(core_axis_name="c", subcore_axis_name="s")

    @functools.partial(
        pl.kernel,
        out_type=jax.ShapeDtypeStruct((CHTOK, D_MODEL), jnp.float32),
        mesh=mesh,
        scratch_types=[
            pltpu.VMEM((CHUNKS, CHUNK), jnp.int32),
            [pltpu.VMEM((CHUNK, D_MODEL), jnp.float32) for _ in range(NBUF)],
            [pltpu.SemaphoreType.DMA for _ in range(NBUF)],
            [pltpu.SemaphoreType.DMA for _ in range(NBUF)],
        ],
    )
    def gather_kernel(idx_hbm, table_hbm, out_hbm, idx_v, bufs, gsem, osem):
        wid = lax.axis_index("s") * 2 + lax.axis_index("c")
        base = wid * ROWS_PER_W
        pltpu.sync_copy(idx_hbm.at[wid], idx_v)

        def gath(j, s):
            pltpu.async_copy(table_hbm.at[idx_v.at[j]], bufs[s], gsem[s])

        def gath_wait(j, s):
            pltpu.make_async_copy(
                table_hbm.at[idx_v.at[j]], bufs[s], gsem[s]
            ).wait()

        def outc(j, s):
            pltpu.async_copy(
                bufs[s], out_hbm.at[pl.ds(base + j * CHUNK, CHUNK)], osem[s]
            )

        def outc_wait(j, s):
            pltpu.make_async_copy(
                bufs[s], out_hbm.at[pl.ds(base + j * CHUNK, CHUNK)], osem[s]
            ).wait()

        def prefetch(x, s):
            if isinstance(x, int) and x < NBUF:
                pass
            else:
                outc_wait(x - NBUF, s)
            gath(x, s)

        def process(j, s):
            gath_wait(j, s)
            outc(j, s)

        for j in range(LOOK):
            prefetch(j, j % NBUF)
        for j in range(LOOK):
            prefetch(j + LOOK, (j + LOOK) % NBUF)
            process(j, j % NBUF)

        G = (CHUNKS - 2 * LOOK) // NBUF
        def body(o, _):
            for k in range(NBUF):
                j = LOOK + o * NBUF + k
                prefetch(j + LOOK, k)
                process(j, (LOOK + k) % NBUF)
            return 0

        lax.fori_loop(0, G, body, 0)

        for j in range(LOOK + G * NBUF, CHUNKS):
            if j + LOOK < CHUNKS:
                prefetch(j + LOOK, (j + LOOK) % NBUF)
            process(j, j % NBUF)
        for j in range(CHUNKS - NBUF, CHUNKS):
            outc_wait(j, j % NBUF)

    return gather_kernel(types32, table2)


def _tc_tail(e2, cont3, Wc, bc2, Wo, bo2, gamma2, beta2, chunk, prev):
    TBLK = 4096
    nblk = CHTOK // TBLK
    grid = (nblk,)

    def body(e_ref, c_ref, wc_ref, bc_ref, wo_ref, bo_ref, g_ref, b_ref,
             *rest):
        o_ref = rest[-1]
        wo_bot = wo_ref[...][HALF:, :]
        w2 = jnp.dot(wc_ref[...], wo_bot, preferred_element_type=jnp.float32)
        b2 = (
            jnp.dot(bc_ref[...], wo_bot, preferred_element_type=jnp.float32)
            + bo_ref[...]
        )
        o = (
            e_ref[...]
            + lax.dot_general(
                c_ref[...],
                w2,
                (((0,), (0,)), ((), ())),
                preferred_element_type=jnp.float32,
            )
            + b2
        )
        mu = jnp.mean(o, axis=-1, keepdims=True)
        d = o - mu
        var = jnp.mean(d * d, axis=-1, keepdims=True)
        y = d * lax.rsqrt(var + 1e-5)
        o_ref[...] = y * g_ref[...] + b_ref[...]

    def wspec(shape):
        return pl.BlockSpec(shape, lambda i: (0, 0))

    in_specs = [
        pl.BlockSpec((TBLK, D_MODEL), lambda i: (i, 0)),
        pl.BlockSpec((3, TBLK), lambda i, c=chunk: (0, i + c * nblk)),
        wspec((3, HALF)),
        wspec((1, HALF)),
        wspec((D_MODEL, D_MODEL)),
        wspec((1, D_MODEL)),
        wspec((1, D_MODEL)),
        wspec((1, D_MODEL)),
    ]
    args = [e2, cont3, Wc, bc2, Wo, bo2, gamma2, beta2]
    io_aliases = {}
    if prev is not None:
        in_specs.append(pl.BlockSpec(memory_space=pl.ANY))
        args.append(prev)
        io_aliases = {8: 0}

    return pl.pallas_call(
        body,
        grid=grid,
        in_specs=in_specs,
        out_specs=pl.BlockSpec(
            (TBLK, D_MODEL), lambda i, c=chunk: (i + c * nblk, 0)
        ),
        out_shape=jax.ShapeDtypeStruct((TOKENS, D_MODEL), jnp.float32),
        input_output_aliases=io_aliases,
    )(*args)


def kernel(types, cont, table, Wc, bc, Wo, bo, gamma, beta):
    types32 = types.astype(jnp.int32).reshape(NCH, NW, CHUNKS, CHUNK)
    table2 = _fuse_table(table.T, Wo[:HALF, :])
    cont3 = jnp.transpose(cont, (2, 0, 1)).reshape(3, TOKENS)
    bc2 = bc.reshape(1, HALF)
    bo2 = bo.reshape(1, D_MODEL)
    gamma2 = gamma.reshape(1, D_MODEL)
    beta2 = beta.reshape(1, D_MODEL)

    e2s = [_sc_gather(types32[c], table2) for c in range(NCH)]
    out = None
    for c in range(NCH):
        out = _tc_tail(
            e2s[c], cont3, Wc, bc2, Wo, bo2, gamma2, beta2, c, out
        )
    return out.reshape(B, L, D_MODEL)
